# baseline (speedup 1.0000x reference)
import functools

import jax
import jax.numpy as jnp
from jax import lax
from jax.experimental import pallas as pl
from jax.experimental.pallas import tpu as pltpu
from jax.experimental.pallas import tpu_sc as plsc

B, N, K, H, V = 8, 128, 8, 32, 512
HP = H // 2
NW = 32
BW = 136


def _splat(x):
    return jnp.full((16,), x, jnp.int32)


def _sc_body(tab_hbm, ed_hbm, bias_hbm, out_hbm, tab_v, idx_v, bias_v,
             slab_v):
    wid = lax.axis_index("s") * 2 + lax.axis_index("c")
    b = wid // 4
    iblk = wid % 4
    pltpu.sync_copy(tab_hbm, tab_v)
    pltpu.sync_copy(bias_hbm.at[b], bias_v)
    lane = lax.iota(jnp.int32, 16)
    eighth = jnp.full((32,), 0.125, jnp.bfloat16)

    def row_body(rr, _):
        r = iblk * 32 + rr + 1
        pltpu.sync_copy(ed_hbm.at[b * N + (r - 1)], idx_v)

        def g_body(g):
            d = [idx_v[pl.ds(k * N + g * 16, 16)] for k in range(K)]
            bias_g = bias_v[pl.ds(r * BW + 1 + g * 16, 16)]

            def hp_body(hp):
                a = [
                    plsc.bitcast(
                        plsc.load_gather(tab_v, [d[k] + hp * V]),
                        jnp.bfloat16)
                    for k in range(K)
                ]
                s = ((a[0] + a[1]) + (a[2] + a[3])) + (
                    (a[4] + a[5]) + (a[6] + a[7]))
                e0, e1 = plsc.unpack(s * eighth,
                                     format=plsc.PackFormat.INTERLEAVED)
                col = lane + (1 + g * 16)
                plsc.store_scatter(slab_v, [_splat(2 * hp), col],
                                   e0 + bias_g)
                plsc.store_scatter(slab_v, [_splat(2 * hp + 1), col],
                                   e1 + bias_g)

            plsc.parallel_loop(0, HP, 1, unroll=4)(hp_body)

        plsc.parallel_loop(0, N // 16, 1)(g_body)

        v0 = plsc.load_gather(bias_v, [_splat(r * BW)])
        plsc.store_scatter(slab_v, [lane, _splat(0)], v0)
        plsc.store_scatter(slab_v, [lane + 16, _splat(0)], v0)
        pltpu.sync_copy(slab_v, out_hbm.at[b, :, r, :])
        return 0

    lax.fori_loop(0, N // 4, row_body, 0)

    @pl.when(iblk == 0)
    def _():
        def h_body(h, _):
            for g in range(N // 16):
                bg = bias_v[pl.ds(g * 16, 16)]
                plsc.store_scatter(slab_v, [_splat(h), lane + g * 16], bg)
            return 0

        lax.fori_loop(0, H, h_body, 0)
        v128 = plsc.load_gather(bias_v, [_splat(N)])
        plsc.store_scatter(slab_v, [lane, _splat(N)], v128)
        plsc.store_scatter(slab_v, [lane + 16, _splat(N)], v128)
        pltpu.sync_copy(slab_v, out_hbm.at[b, :, 0, :])


@jax.jit
def _sc_full(tab_packed, ed_rows, bias2):
    mesh = plsc.VectorSubcoreMesh(core_axis_name="c", subcore_axis_name="s")
    return pl.kernel(
        _sc_body,
        out_type=jax.ShapeDtypeStruct((B, H, N + 1, N + 1), jnp.float32),
        mesh=mesh,
        compiler_params=pltpu.CompilerParams(needs_layout_passes=False),
        scratch_types=[
            pltpu.VMEM((HP * V,), jnp.int32),
            pltpu.VMEM((N * K,), jnp.int32),
            pltpu.VMEM(((N + 1) * BW,), jnp.float32),
            pltpu.VMEM((H, N + 1), jnp.float32),
        ],
    )(tab_packed, ed_rows, bias2)


def kernel(attn_bias, edge_data, edge_encoder_weight):
    ed = jnp.transpose(edge_data.astype(jnp.int32), (0, 1, 3, 2)).reshape(
        B * N, K * N)
    w16 = lax.bitcast_convert_type(
        edge_encoder_weight.astype(jnp.bfloat16), jnp.uint16
    ).astype(jnp.uint32)
    packed = (w16[:, 0::2] | (w16[:, 1::2] << 16)).astype(jnp.int32)
    tab = packed.T.reshape(HP * V)
    bias2 = jnp.pad(2.0 * attn_bias,
                    ((0, 0), (0, 0), (0, BW - (N + 1)))).reshape(
                        B, (N + 1) * BW)
    return _sc_full(tab, ed, bias2)


# speedup vs baseline: 1.3319x; 1.3319x over previous
import functools

import jax
import jax.numpy as jnp
from jax import lax
from jax.experimental import pallas as pl
from jax.experimental.pallas import tpu as pltpu
from jax.experimental.pallas import tpu_sc as plsc

B, N, K, H, V = 8, 128, 8, 32, 512
HP = H // 2
NW = 32
BW = 136


def _splat(x):
    return jnp.full((16,), x, jnp.int32)


def _sc_body(tab_hbm, ed_hbm, bias_hbm, out_hbm, tab_v, idx_v, bias_v,
             slab_v, sin0, sin1, sout0, sout1):
    wid = lax.axis_index("s") * 2 + lax.axis_index("c")
    b = wid // 4
    iblk = wid % 4
    pltpu.sync_copy(tab_hbm, tab_v)
    pltpu.sync_copy(bias_hbm.at[b], bias_v)
    lane = lax.iota(jnp.int32, 16)
    eighth = jnp.full((32,), 0.125, jnp.bfloat16)
    sin = (sin0, sin1)
    sout = (sout0, sout1)
    row0 = b * N + iblk * 32

    pltpu.make_async_copy(ed_hbm.at[row0], idx_v.at[0], sin[0]).start()

    def compute_row(rr, p):
        r = iblk * 32 + rr + 1
        slab = slab_v.at[p]

        def g_body(g):
            d = [idx_v[p, pl.ds(k * N + g * 16, 16)] for k in range(K)]
            bias_g = bias_v[pl.ds(r * BW + 1 + g * 16, 16)]

            def hp_body(hp):
                a = [
                    plsc.bitcast(
                        plsc.load_gather(tab_v, [d[k] + hp * V]),
                        jnp.bfloat16)
                    for k in range(K)
                ]
                s = ((a[0] + a[1]) + (a[2] + a[3])) + (
                    (a[4] + a[5]) + (a[6] + a[7]))
                e0, e1 = plsc.unpack(s * eighth,
                                     format=plsc.PackFormat.INTERLEAVED)
                col = lane + (1 + g * 16)
                plsc.store_scatter(slab, [_splat(2 * hp), col],
                                   e0 + bias_g)
                plsc.store_scatter(slab, [_splat(2 * hp + 1), col],
                                   e1 + bias_g)

            plsc.parallel_loop(0, HP, 1, unroll=4)(hp_body)

        plsc.parallel_loop(0, N // 16, 1)(g_body)

        v0 = plsc.load_gather(bias_v, [_splat(r * BW)])
        plsc.store_scatter(slab, [lane, _splat(0)], v0)
        plsc.store_scatter(slab, [lane + 16, _splat(0)], v0)

    def pair_body(t, _):
        for p in range(2):
            rr = 2 * t + p
            r = iblk * 32 + rr + 1
            pltpu.make_async_copy(
                ed_hbm.at[row0 + rr], idx_v.at[p], sin[p]).wait()
            if p == 0:
                pltpu.make_async_copy(
                    ed_hbm.at[row0 + rr + 1], idx_v.at[1], sin[1]).start()
            else:
                @pl.when(t < N // 8 - 1)
                def _():
                    pltpu.make_async_copy(
                        ed_hbm.at[row0 + rr + 1], idx_v.at[0],
                        sin[0]).start()

            @pl.when(t >= 1)
            def _():
                pltpu.make_async_copy(
                    slab_v.at[p], out_hbm.at[b, :, r - 2, :],
                    sout[p]).wait()

            compute_row(rr, p)
            pltpu.make_async_copy(
                slab_v.at[p], out_hbm.at[b, :, r, :], sout[p]).start()
        return 0

    lax.fori_loop(0, N // 8, pair_body, 0)
    for p in range(2):
        pltpu.make_async_copy(
            slab_v.at[p], out_hbm.at[b, :, iblk * 32 + 31 + p, :],
            sout[p]).wait()

    @pl.when(iblk == 0)
    def _():
        def h_body(h, _):
            for g in range(N // 16):
                bg = bias_v[pl.ds(g * 16, 16)]
                plsc.store_scatter(slab_v.at[0],
                                   [_splat(h), lane + g * 16], bg)
            return 0

        lax.fori_loop(0, H, h_body, 0)
        v128 = plsc.load_gather(bias_v, [_splat(N)])
        plsc.store_scatter(slab_v.at[0], [lane, _splat(N)], v128)
        plsc.store_scatter(slab_v.at[0], [lane + 16, _splat(N)], v128)
        pltpu.sync_copy(slab_v.at[0], out_hbm.at[b, :, 0, :])


@jax.jit
def _sc_full(tab_packed, ed_rows, bias2):
    mesh = plsc.VectorSubcoreMesh(core_axis_name="c", subcore_axis_name="s")
    return pl.kernel(
        _sc_body,
        out_type=jax.ShapeDtypeStruct((B, H, N + 1, N + 1), jnp.float32),
        mesh=mesh,
        compiler_params=pltpu.CompilerParams(needs_layout_passes=False),
        scratch_types=[
            pltpu.VMEM((HP * V,), jnp.int32),
            pltpu.VMEM((2, N * K), jnp.int32),
            pltpu.VMEM(((N + 1) * BW,), jnp.float32),
            pltpu.VMEM((2, H, N + 1), jnp.float32),
            pltpu.SemaphoreType.DMA,
            pltpu.SemaphoreType.DMA,
            pltpu.SemaphoreType.DMA,
            pltpu.SemaphoreType.DMA,
        ],
    )(tab_packed, ed_rows, bias2)


def kernel(attn_bias, edge_data, edge_encoder_weight):
    ed = jnp.transpose(edge_data.astype(jnp.int32), (0, 1, 3, 2)).reshape(
        B * N, K * N)
    w16 = lax.bitcast_convert_type(
        edge_encoder_weight.astype(jnp.bfloat16), jnp.uint16
    ).astype(jnp.uint32)
    packed = (w16[:, 0::2] | (w16[:, 1::2] << 16)).astype(jnp.int32)
    tab = packed.T.reshape(HP * V)
    bias2 = jnp.pad(2.0 * attn_bias,
                    ((0, 0), (0, 0), (0, BW - (N + 1)))).reshape(
                        B, (N + 1) * BW)
    return _sc_full(tab, ed, bias2)
